# baseline (speedup 1.0000x reference)
import functools

import jax
import jax.numpy as jnp
from jax import lax
from jax.experimental import pallas as pl
from jax.experimental.pallas import tpu as pltpu
from jax.experimental.pallas import tpu_sc as plsc

N = 10000
E = 320000
D = 128
NPAD = 10240
K = 8000
NC, NS = 2, 16
NW = NC * NS
CH = 128
NCHUNK = E // CH
ROWS_PT = NPAD // NS

_mesh = plsc.VectorSubcoreMesh(core_axis_name="c", subcore_axis_name="s")


EPAD = 2560 * CH
DH = D // NC
NFIX = (EPAD // CH) // NS
EPT = NFIX * CH


def _agg_body(src_p, dst_p, xh_hbm, zrow_hbm, zcnt_hbm, ones_hbm,
              agg_out, cnt_out, acc, cacc, sidx_f,
              rows0, rows1, rows2, rows3, d0, d1, d2, d3, ones_v,
              sg0, sg1, sg2, sg3, si0, si1, si2, si3,
              so0, so1, so2, so3, sr0, sr1, sr2, sr3):
  c = lax.axis_index("c")
  s = lax.axis_index("s")
  row0 = s * ROWS_PT
  base = s * EPT
  dbufs = (d0, d1, d2, d3)
  isems = (si0, si1, si2, si3)
  osems = (so0, so1, so2, so3)
  rsems = (sr0, sr1, sr2, sr3)
  rbufs = (rows0, rows1, rows2, rows3)
  gsems = (sg0, sg1, sg2, sg3)
  xc = xh_hbm.at[c]
  pltpu.sync_copy(zrow_hbm, acc.at[pl.ds(row0, ROWS_PT)])
  pltpu.sync_copy(zcnt_hbm, cacc.at[pl.ds(row0, ROWS_PT)])
  pltpu.sync_copy(ones_hbm, ones_v)
  pltpu.sync_copy(src_p.at[pl.ds(base, EPT)], sidx_f)
  plsc.subcore_barrier()

  def gidx(i):
    return sidx_f.at[pl.ds(i * CH, CH)]

  def fire_gather(i, b4):
    pltpu.async_copy(xc.at[gidx(i)], rbufs[b4], gsems[b4])

  def fire_didx(i, b4):
    pltpu.async_copy(dst_p.at[pl.ds(base + i * CH, CH)], dbufs[b4],
                     isems[b4])

  fire_didx(0, 0)
  fire_didx(1, 1)
  fire_gather(0, 0)
  fire_gather(1, 1)

  def group(j, carry):
    for b in range(4):
      i = 4 * j + b
      rbuf, dbuf = rbufs[b], dbufs[b]
      pltpu.make_async_copy(dst_p.at[pl.ds(base, CH)], dbuf, isems[b]).wait()
      pltpu.make_async_copy(xc.at[gidx(0)], rbuf, gsems[b]).wait()
      pltpu.async_copy(rbuf, acc.at[dbuf], rsems[b], add=True)

      @pl.when(c == 0)
      def _():
        pltpu.async_copy(ones_v, cacc.at[dbuf], osems[b], add=True)

      @pl.when(i + 2 < NFIX)
      def _():
        b4 = (b + 2) % 4

        @pl.when(i >= 2)
        def _():
          pltpu.make_async_copy(rbufs[b4], acc.at[d0], rsems[b4]).wait()

          @pl.when(c == 0)
          def _():
            pltpu.make_async_copy(ones_v, cacc.at[d0], osems[b4]).wait()

        fire_didx(i + 2, b4)
        fire_gather(i + 2, b4)

    return carry

  lax.fori_loop(0, NFIX // 4, group, jnp.int32(0))
  for b4 in range(4):
    pltpu.make_async_copy(rbufs[b4], acc.at[d0], rsems[b4]).wait()

    @pl.when(c == 0)
    def _():
      pltpu.make_async_copy(ones_v, cacc.at[d0], osems[b4]).wait()

  plsc.subcore_barrier()
  pltpu.sync_copy(acc.at[pl.ds(row0, ROWS_PT)],
                  agg_out.at[c, pl.ds(row0, ROWS_PT)])

  @pl.when(c == 0)
  def _():
    pltpu.sync_copy(cacc.at[pl.ds(row0, ROWS_PT)],
                    cnt_out.at[pl.ds(row0, ROWS_PT)])


_stage_a = functools.partial(
    pl.kernel,
    out_type=(jax.ShapeDtypeStruct((NC, NPAD, DH), jnp.float32),
              jax.ShapeDtypeStruct((NPAD,), jnp.float32)),
    mesh=_mesh,
    compiler_params=pltpu.CompilerParams(use_tc_tiling_on_sc=False),
    scratch_types=[
        pltpu.VMEM_SHARED((NPAD, DH), jnp.float32),
        pltpu.VMEM_SHARED((NPAD,), jnp.float32),
        pltpu.VMEM((EPT,), jnp.int32),
        pltpu.VMEM((CH, DH), jnp.float32),
        pltpu.VMEM((CH, DH), jnp.float32),
        pltpu.VMEM((CH, DH), jnp.float32),
        pltpu.VMEM((CH, DH), jnp.float32),
        pltpu.VMEM((CH,), jnp.int32),
        pltpu.VMEM((CH,), jnp.int32),
        pltpu.VMEM((CH,), jnp.int32),
        pltpu.VMEM((CH,), jnp.int32),
        pltpu.VMEM((CH,), jnp.float32),
    ] + [pltpu.SemaphoreType.DMA] * 16,
)(_agg_body)


NFIXC = (EPAD // CH) // NW
EPTC = NFIXC * CH


def _seg1_body(src_p, dst_p, r_hbm, zcnt_hbm,
               s2_out, sacc, r_local, sidx_f,
               vals0, vals1, d0, d1, d2, d3,
               si0, si1, si2, si3, ss0, ss1, ss2, ss3):
  c = lax.axis_index("c")
  s = lax.axis_index("s")
  wid = s * NC + c
  row0 = s * ROWS_PT
  base = wid * EPTC
  dbufs = (d0, d1, d2, d3)
  isems = (si0, si1, si2, si3)
  ssems = (ss0, ss1, ss2, ss3)
  vbufs = (vals0, vals1)
  pltpu.sync_copy(zcnt_hbm, sacc.at[pl.ds(row0, ROWS_PT)])
  pltpu.sync_copy(r_hbm, r_local)
  pltpu.sync_copy(src_p.at[pl.ds(base, EPTC)], sidx_f)

  def fire_didx(i, b4):
    pltpu.async_copy(dst_p.at[pl.ds(base + i * CH, CH)], dbufs[b4],
                     isems[b4])

  fire_didx(0, 0)
  fire_didx(1, 1)
  plsc.subcore_barrier()

  def fill(i, vbuf):
    for b in range(CH // 16):
      vs = sidx_f[pl.ds(i * CH + b * 16, 16)]
      vbuf[pl.ds(b * 16, 16)] = plsc.load_gather(r_local, [vs])

  def group(j, carry):
    for b in range(4):
      i = 4 * j + b
      vbuf = vbufs[b % 2]
      pltpu.make_async_copy(dst_p.at[pl.ds(base, CH)], dbufs[b],
                            isems[b]).wait()

      @pl.when(i >= 2)
      def _():
        pltpu.make_async_copy(vbuf, sacc.at[d0], ssems[(b + 2) % 4]).wait()

      fill(i, vbuf)
      pltpu.async_copy(vbuf, sacc.at[dbufs[b]], ssems[b], add=True)

      @pl.when(i + 2 < NFIXC)
      def _():
        fire_didx(i + 2, (b + 2) % 4)

    return carry

  lax.fori_loop(0, NFIXC // 4, group, jnp.int32(0))
  pltpu.make_async_copy(vals0, sacc.at[d0], ssems[(NFIXC - 2) % 4]).wait()
  pltpu.make_async_copy(vals1, sacc.at[d0], ssems[(NFIXC - 1) % 4]).wait()
  plsc.subcore_barrier()
  pltpu.sync_copy(sacc.at[pl.ds(row0, ROWS_PT)],
                  s2_out.at[c, pl.ds(row0, ROWS_PT)])


_stage_c = functools.partial(
    pl.kernel,
    out_type=jax.ShapeDtypeStruct((NC, NPAD), jnp.float32),
    mesh=_mesh,
    compiler_params=pltpu.CompilerParams(needs_layout_passes=False),
    scratch_types=[
        pltpu.VMEM_SHARED((NPAD,), jnp.float32),
        pltpu.VMEM((NPAD,), jnp.float32),
        pltpu.VMEM((EPTC,), jnp.int32),
        pltpu.VMEM((CH,), jnp.float32),
        pltpu.VMEM((CH,), jnp.float32),
        pltpu.VMEM((CH,), jnp.int32),
        pltpu.VMEM((CH,), jnp.int32),
        pltpu.VMEM((CH,), jnp.int32),
        pltpu.VMEM((CH,), jnp.int32),
        pltpu.SemaphoreType.DMA,
        pltpu.SemaphoreType.DMA,
        pltpu.SemaphoreType.DMA,
        pltpu.SemaphoreType.DMA,
        pltpu.SemaphoreType.DMA,
        pltpu.SemaphoreType.DMA,
        pltpu.SemaphoreType.DMA,
        pltpu.SemaphoreType.DMA,
    ],
)(_seg1_body)


BN = 1024


def _h_body(agg_ref, cntb_ref, x_ref, wl_ref, bl_ref, wr_ref, wcat_ref,
            h_ref, rrgt_ref):
  a = jnp.concatenate([agg_ref[0], agg_ref[1]], axis=1)
  mean = a / jnp.maximum(cntb_ref[...], 1.0)
  h = jnp.dot(mean, wl_ref[...], preferred_element_type=jnp.float32)
  h += jnp.dot(x_ref[...], wr_ref[...], preferred_element_type=jnp.float32)
  h = jnp.maximum(h + bl_ref[...], 0.0)
  h_ref[...] = h
  rrgt_ref[...] = lax.dot_general(
      wcat_ref[...], h, (((0,), (1,)), ((), ())),
      preferred_element_type=jnp.float32)


def _stage_b(aggp, cntb, xpad, wl, bl, wr, wcat):
  return pl.pallas_call(
      _h_body,
      grid=(NPAD // BN,),
      in_specs=[
          pl.BlockSpec((NC, BN, DH), lambda i: (0, i, 0)),
          pl.BlockSpec((BN, D), lambda i: (i, 0)),
          pl.BlockSpec((BN, D), lambda i: (i, 0)),
          pl.BlockSpec((D, D), lambda i: (0, 0)),
          pl.BlockSpec((1, D), lambda i: (0, 0)),
          pl.BlockSpec((D, D), lambda i: (0, 0)),
          pl.BlockSpec((D, D), lambda i: (0, 0)),
      ],
      out_specs=[
          pl.BlockSpec((BN, D), lambda i: (i, 0)),
          pl.BlockSpec((D, BN), lambda i: (0, i)),
      ],
      out_shape=[
          jax.ShapeDtypeStruct((NPAD, D), jnp.float32),
          jax.ShapeDtypeStruct((D, NPAD), jnp.float32),
      ],
  )(aggp, cntb, xpad, wl, bl, wr, wcat)


NROW = NPAD // 128


def _readout_body(s2p_ref, rho_ref, g_ref, h_ref, brel_ref, bgate_ref,
                  out_ref, coef_ref):
  s2 = s2p_ref[0] + s2p_ref[1]
  score = jnp.tanh(s2 + rho_ref[...] + brel_ref[...])
  ub = lax.bitcast_convert_type(score, jnp.uint32)
  sgn = ub >> jnp.uint32(31)
  flip = jnp.where(sgn == jnp.uint32(1),
                   jnp.uint32(0xFFFFFFFF), jnp.uint32(0x80000000))
  key = ub ^ flip
  rows = lax.broadcasted_iota(jnp.int32, (NROW, 128), 0)
  cols = lax.broadcasted_iota(jnp.int32, (NROW, 128), 1)
  idx = rows * 128 + cols
  key = jnp.where(idx < N, key, jnp.uint32(0))

  def cnt_ge(m):
    return jnp.sum((key >= m).astype(jnp.int32))

  def bit_body(i, t):
    cand = t | (jnp.uint32(1) << (jnp.uint32(31) - i.astype(jnp.uint32)))
    return jnp.where(cnt_ge(cand) >= K, cand, t)

  t = lax.fori_loop(0, 32, bit_body, jnp.uint32(0))
  c_gt = cnt_ge(t + jnp.uint32(1))
  r_extra = K - c_gt
  ties = key == t

  def tie_cnt(j):
    return jnp.sum((ties & (idx <= j)).astype(jnp.int32))

  def bs_body(i, lohi):
    lo, hi = lohi
    mid = (lo + hi) // 2
    pred = tie_cnt(mid) >= r_extra
    return (jnp.where(pred, lo, mid + 1), jnp.where(pred, mid, hi))

  jstar, _ = lax.fori_loop(0, 14, bs_body,
                           (jnp.int32(0), jnp.int32(NPAD - 1)))
  sel = (key > t) | (ties & (idx <= jstar))

  gate = score * g_ref[...] + bgate_ref[...]
  gmax = jnp.max(jnp.where(sel, gate, -1e30))
  e = jnp.where(sel, jnp.exp(gate - gmax), 0.0)
  z = jnp.sum(e)
  coef_ref[...] = e * score / z

  def mv(rr, acc):
    crow = coef_ref[pl.ds(rr, 1), :]
    hblk = h_ref[pl.ds(rr * 128, 128), :]
    return acc + jnp.dot(crow, hblk, preferred_element_type=jnp.float32)

  out_ref[...] = lax.fori_loop(0, NROW, mv, jnp.zeros((1, D), jnp.float32))


def _stage_d(s2p3, rho2d, g2d, h, brelb, bgateb):
  return pl.pallas_call(
      _readout_body,
      out_shape=jax.ShapeDtypeStruct((1, D), jnp.float32),
      scratch_shapes=[pltpu.VMEM((NROW, 128), jnp.float32)],
  )(s2p3, rho2d, g2d, h, brelb, bgateb)


@jax.jit
def kernel(x, edge_index, W_l, b_l, W_r, W_rel, b_rel, W_root, W_gate,
           b_gate):
  src = edge_index[0]
  dst = edge_index[1]
  npad_e = EPAD - E
  fill = jnp.arange(npad_e, dtype=jnp.int32)
  src_p = jnp.concatenate([src, fill % N])
  dst_p = jnp.concatenate([dst, N + fill % (NPAD - N)])
  zrow = jnp.zeros((ROWS_PT, DH), jnp.float32)
  zcnt = jnp.zeros((ROWS_PT,), jnp.float32)
  ones = jnp.ones((CH,), jnp.float32)
  xh = x.reshape(N, NC, DH).transpose(1, 0, 2)

  aggp, cnts = _stage_a(src_p, dst_p, xh, zrow, zcnt, ones)

  cntb = jnp.broadcast_to(cnts[:, None], (NPAD, D))
  xpad = jnp.pad(x, ((0, NPAD - N), (0, 0)))
  wcat = jnp.concatenate(
      [W_rel, W_root, W_gate, jnp.zeros((D, D - 3), jnp.float32)], axis=1)
  h, rrgt = _stage_b(aggp, cntb, xpad, W_l, b_l.reshape(1, D), W_r, wcat)

  r = rrgt[0]
  s2p = _stage_c(src_p, dst_p, r, zcnt)

  s2p3 = s2p.reshape(NC, NROW, 128)
  rho2d = rrgt[1].reshape(NROW, 128)
  g2d = rrgt[2].reshape(NROW, 128)
  brelb = jnp.broadcast_to(b_rel.reshape(1, 1), (1, 128))
  bgateb = jnp.broadcast_to(b_gate.reshape(1, 1), (1, 128))
  return _stage_d(s2p3, rho2d, g2d, h, brelb, bgateb)


# speedup vs baseline: 1.1149x; 1.1149x over previous
import functools

import jax
import jax.numpy as jnp
from jax import lax
from jax.experimental import pallas as pl
from jax.experimental.pallas import tpu as pltpu
from jax.experimental.pallas import tpu_sc as plsc

N = 10000
E = 320000
D = 128
NPAD = 10240
K = 8000
NC, NS = 2, 16
NW = NC * NS
CH = 128
NCHUNK = E // CH
ROWS_PT = NPAD // NS

_mesh = plsc.VectorSubcoreMesh(core_axis_name="c", subcore_axis_name="s")


EPAD = 2560 * CH
DH = D // NC
G = 4
CHK = G * CH
NCH3 = EPAD // CHK
NFA = NCH3 // NS


def _agg_body(src2, dst2, xh_hbm, zrow_hbm, zcnt_hbm, ones_hbm,
              agg_out, cnt_out, acc, cacc,
              rows0, rows1, s0, s1, d0, d1, d2, d3, ones_v,
              sg0, sg1, sl0, sl1, si0, si1, si2, si3,
              so0, so1, so2, so3):
  c = lax.axis_index("c")
  s = lax.axis_index("s")
  row0 = s * ROWS_PT
  base = s * NFA
  dbufs = (d0, d1, d2, d3)
  sbufs = (s0, s1)
  rbufs = (rows0, rows1)
  isems = (si0, si1, si2, si3)
  osems = (so0, so1, so2, so3)
  lsems = (sl0, sl1)
  gsems = (sg0, sg1)
  xc = xh_hbm.at[c]
  pltpu.sync_copy(zrow_hbm, acc.at[pl.ds(row0, ROWS_PT)])
  pltpu.sync_copy(zcnt_hbm, cacc.at[pl.ds(row0, ROWS_PT)])
  pltpu.sync_copy(ones_hbm, ones_v)
  plsc.subcore_barrier()

  def fire_sidx(i, b2):
    pltpu.async_copy(src2.at[base + i], sbufs[b2], lsems[b2])

  def fire_didx(i, b4):
    pltpu.async_copy(dst2.at[base + i], dbufs[b4], isems[b4])

  def fire_gather(i, b2):
    pltpu.async_copy(xc.at[sbufs[b2]], rbufs[b2], gsems[b2])

  for b in range(2):
    fire_sidx(b, b)
    fire_didx(b, b)
  for b in range(2):
    pltpu.make_async_copy(src2.at[base], sbufs[b], lsems[b]).wait()
    fire_gather(b, b)

  def group(j, carry):
    for b in range(4):
      i = 4 * j + b
      b2 = b % 2
      rbuf, dbuf = rbufs[b2], dbufs[b]
      pltpu.make_async_copy(dst2.at[base], dbuf, isems[b]).wait()
      pltpu.make_async_copy(xc.at[sbufs[b2]], rbuf, gsems[b2]).wait()
      pltpu.sync_copy(rbuf, acc.at[dbuf], add=True)

      @pl.when(c == 0)
      def _():
        pltpu.async_copy(ones_v, cacc.at[dbuf], osems[b], add=True)

      @pl.when(i + 2 < NFA)
      def _():
        b4 = (b + 2) % 4

        @pl.when((i >= 2) & (c == 0))
        def _():
          pltpu.make_async_copy(ones_v, cacc.at[d0], osems[b4]).wait()

        fire_didx(i + 2, b4)
        fire_sidx(i + 2, b2)
        pltpu.make_async_copy(src2.at[base], sbufs[b2], lsems[b2]).wait()
        fire_gather(i + 2, b2)

    return carry

  lax.fori_loop(0, NFA // 4, group, jnp.int32(0))

  @pl.when(c == 0)
  def _():
    for b4 in range(4):
      pltpu.make_async_copy(ones_v, cacc.at[d0], osems[b4]).wait()

  plsc.subcore_barrier()
  pltpu.sync_copy(acc.at[pl.ds(row0, ROWS_PT)],
                  agg_out.at[c, pl.ds(row0, ROWS_PT)])

  @pl.when(c == 0)
  def _():
    pltpu.sync_copy(cacc.at[pl.ds(row0, ROWS_PT)],
                    cnt_out.at[pl.ds(row0, ROWS_PT)])


_stage_a = functools.partial(
    pl.kernel,
    out_type=(jax.ShapeDtypeStruct((NC, NPAD, DH), jnp.float32),
              jax.ShapeDtypeStruct((NPAD,), jnp.float32)),
    mesh=_mesh,
    compiler_params=pltpu.CompilerParams(use_tc_tiling_on_sc=False),
    scratch_types=[
        pltpu.VMEM_SHARED((NPAD, DH), jnp.float32),
        pltpu.VMEM_SHARED((NPAD,), jnp.float32),
        pltpu.VMEM((CHK, DH), jnp.float32),
        pltpu.VMEM((CHK, DH), jnp.float32),
        pltpu.VMEM((CHK,), jnp.int32),
        pltpu.VMEM((CHK,), jnp.int32),
        pltpu.VMEM((CHK,), jnp.int32),
        pltpu.VMEM((CHK,), jnp.int32),
        pltpu.VMEM((CHK,), jnp.int32),
        pltpu.VMEM((CHK,), jnp.int32),
        pltpu.VMEM((CHK,), jnp.float32),
    ] + [pltpu.SemaphoreType.DMA] * 12,
)(_agg_body)


NFC = NCH3 // NW


def _seg1_body(src2, dst2, r_hbm, zcnt_hbm,
               s2_out, sacc, r_local,
               vals0, vals1, s0, s1, d0, d1, d2, d3,
               sl0, sl1, si0, si1, si2, si3, sc0, sc1, sc2, sc3):
  c = lax.axis_index("c")
  s = lax.axis_index("s")
  wid = s * NC + c
  row0 = s * ROWS_PT
  base = wid * NFC
  sbufs = (s0, s1)
  dbufs = (d0, d1, d2, d3)
  vbufs = (vals0, vals1)
  lsems = (sl0, sl1)
  isems = (si0, si1, si2, si3)
  csems = (sc0, sc1, sc2, sc3)
  pltpu.sync_copy(zcnt_hbm, sacc.at[pl.ds(row0, ROWS_PT)])
  pltpu.sync_copy(r_hbm, r_local)

  def fire_sidx(i, b2):
    pltpu.async_copy(src2.at[base + i], sbufs[b2], lsems[b2])

  def fire_didx(i, b4):
    pltpu.async_copy(dst2.at[base + i], dbufs[b4], isems[b4])

  for b in range(2):
    fire_sidx(b, b)
    fire_didx(b, b)
  plsc.subcore_barrier()

  def fill(vbuf, sbuf):
    for k in range(CHK // 16):
      vs = sbuf[pl.ds(k * 16, 16)]
      vbuf[pl.ds(k * 16, 16)] = plsc.load_gather(r_local, [vs])

  def group(j, carry):
    for b in range(4):
      i = 4 * j + b
      b2 = b % 2
      vbuf, sbuf = vbufs[b2], sbufs[b2]
      pltpu.make_async_copy(src2.at[base], sbuf, lsems[b2]).wait()
      pltpu.make_async_copy(dst2.at[base], dbufs[b], isems[b]).wait()

      @pl.when(i >= 2)
      def _():
        pltpu.make_async_copy(vbuf, sacc.at[d0], csems[(b + 2) % 4]).wait()

      fill(vbuf, sbuf)

      @pl.when(i + 2 < NFC)
      def _():
        fire_sidx(i + 2, b2)
        fire_didx(i + 2, (b + 2) % 4)

      pltpu.async_copy(vbuf, sacc.at[dbufs[b]], csems[b], add=True)
    return carry

  lax.fori_loop(0, NFC // 4, group, jnp.int32(0))
  pltpu.make_async_copy(vals0, sacc.at[d0], csems[(NFC - 2) % 4]).wait()
  pltpu.make_async_copy(vals1, sacc.at[d0], csems[(NFC - 1) % 4]).wait()
  plsc.subcore_barrier()
  pltpu.sync_copy(sacc.at[pl.ds(row0, ROWS_PT)],
                  s2_out.at[c, pl.ds(row0, ROWS_PT)])


_stage_c = functools.partial(
    pl.kernel,
    out_type=jax.ShapeDtypeStruct((NC, NPAD), jnp.float32),
    mesh=_mesh,
    compiler_params=pltpu.CompilerParams(needs_layout_passes=False),
    scratch_types=[
        pltpu.VMEM_SHARED((NPAD,), jnp.float32),
        pltpu.VMEM((NPAD,), jnp.float32),
        pltpu.VMEM((CHK,), jnp.float32),
        pltpu.VMEM((CHK,), jnp.float32),
        pltpu.VMEM((CHK,), jnp.int32),
        pltpu.VMEM((CHK,), jnp.int32),
        pltpu.VMEM((CHK,), jnp.int32),
        pltpu.VMEM((CHK,), jnp.int32),
        pltpu.VMEM((CHK,), jnp.int32),
        pltpu.VMEM((CHK,), jnp.int32),
    ] + [pltpu.SemaphoreType.DMA] * 10,
)(_seg1_body)


BN = 1024


def _h_body(agg_ref, cntb_ref, x_ref, wl_ref, bl_ref, wr_ref, wcat_ref,
            h_ref, rrgt_ref):
  a = jnp.concatenate([agg_ref[0], agg_ref[1]], axis=1)
  mean = a / jnp.maximum(cntb_ref[...], 1.0)
  h = jnp.dot(mean, wl_ref[...], preferred_element_type=jnp.float32)
  h += jnp.dot(x_ref[...], wr_ref[...], preferred_element_type=jnp.float32)
  h = jnp.maximum(h + bl_ref[...], 0.0)
  h_ref[...] = h
  rrgt_ref[...] = lax.dot_general(
      wcat_ref[...], h, (((0,), (1,)), ((), ())),
      preferred_element_type=jnp.float32)


def _stage_b(aggp, cntb, xpad, wl, bl, wr, wcat):
  return pl.pallas_call(
      _h_body,
      grid=(NPAD // BN,),
      in_specs=[
          pl.BlockSpec((NC, BN, DH), lambda i: (0, i, 0)),
          pl.BlockSpec((BN, D), lambda i: (i, 0)),
          pl.BlockSpec((BN, D), lambda i: (i, 0)),
          pl.BlockSpec((D, D), lambda i: (0, 0)),
          pl.BlockSpec((1, D), lambda i: (0, 0)),
          pl.BlockSpec((D, D), lambda i: (0, 0)),
          pl.BlockSpec((D, D), lambda i: (0, 0)),
      ],
      out_specs=[
          pl.BlockSpec((BN, D), lambda i: (i, 0)),
          pl.BlockSpec((D, BN), lambda i: (0, i)),
      ],
      out_shape=[
          jax.ShapeDtypeStruct((NPAD, D), jnp.float32),
          jax.ShapeDtypeStruct((D, NPAD), jnp.float32),
      ],
  )(aggp, cntb, xpad, wl, bl, wr, wcat)


NROW = NPAD // 128


def _readout_body(s2p_ref, rho_ref, g_ref, h_ref, brel_ref, bgate_ref,
                  out_ref, coef_ref):
  s2 = s2p_ref[0] + s2p_ref[1]
  score = jnp.tanh(s2 + rho_ref[...] + brel_ref[...])
  ub = lax.bitcast_convert_type(score, jnp.uint32)
  sgn = ub >> jnp.uint32(31)
  flip = jnp.where(sgn == jnp.uint32(1),
                   jnp.uint32(0xFFFFFFFF), jnp.uint32(0x80000000))
  key = ub ^ flip
  rows = lax.broadcasted_iota(jnp.int32, (NROW, 128), 0)
  cols = lax.broadcasted_iota(jnp.int32, (NROW, 128), 1)
  idx = rows * 128 + cols
  key = jnp.where(idx < N, key, jnp.uint32(0))

  def cnt_ge(m):
    return jnp.sum((key >= m).astype(jnp.int32))

  def bit_body(i, t):
    cand = t | (jnp.uint32(1) << (jnp.uint32(31) - i.astype(jnp.uint32)))
    return jnp.where(cnt_ge(cand) >= K, cand, t)

  t = lax.fori_loop(0, 32, bit_body, jnp.uint32(0))
  c_gt = cnt_ge(t + jnp.uint32(1))
  r_extra = K - c_gt
  ties = key == t

  def tie_cnt(j):
    return jnp.sum((ties & (idx <= j)).astype(jnp.int32))

  def bs_body(i, lohi):
    lo, hi = lohi
    mid = (lo + hi) // 2
    pred = tie_cnt(mid) >= r_extra
    return (jnp.where(pred, lo, mid + 1), jnp.where(pred, mid, hi))

  jstar, _ = lax.fori_loop(0, 14, bs_body,
                           (jnp.int32(0), jnp.int32(NPAD - 1)))
  sel = (key > t) | (ties & (idx <= jstar))

  gate = score * g_ref[...] + bgate_ref[...]
  gmax = jnp.max(jnp.where(sel, gate, -1e30))
  e = jnp.where(sel, jnp.exp(gate - gmax), 0.0)
  z = jnp.sum(e)
  coef_ref[...] = e * score / z

  def mv(rr, acc):
    crow = coef_ref[pl.ds(rr, 1), :]
    hblk = h_ref[pl.ds(rr * 128, 128), :]
    return acc + jnp.dot(crow, hblk, preferred_element_type=jnp.float32)

  out_ref[...] = lax.fori_loop(0, NROW, mv, jnp.zeros((1, D), jnp.float32))


def _stage_d(s2p3, rho2d, g2d, h, brelb, bgateb):
  return pl.pallas_call(
      _readout_body,
      out_shape=jax.ShapeDtypeStruct((1, D), jnp.float32),
      scratch_shapes=[pltpu.VMEM((NROW, 128), jnp.float32)],
  )(s2p3, rho2d, g2d, h, brelb, bgateb)


@jax.jit
def kernel(x, edge_index, W_l, b_l, W_r, W_rel, b_rel, W_root, W_gate,
           b_gate):
  src = edge_index[0]
  dst = edge_index[1]
  npad_e = EPAD - E
  fill = jnp.arange(npad_e, dtype=jnp.int32)
  src2 = jnp.concatenate([src, fill % N]).reshape(NCH3, CHK)
  dst2 = jnp.concatenate([dst, N + fill % (NPAD - N)]).reshape(NCH3, CHK)
  zrow = jnp.zeros((ROWS_PT, DH), jnp.float32)
  zcnt = jnp.zeros((ROWS_PT,), jnp.float32)
  ones = jnp.ones((CHK,), jnp.float32)
  xh = x.reshape(N, NC, DH).transpose(1, 0, 2)

  aggp, cnts = _stage_a(src2, dst2, xh, zrow, zcnt, ones)

  cntb = jnp.broadcast_to(cnts[:, None], (NPAD, D))
  xpad = jnp.pad(x, ((0, NPAD - N), (0, 0)))
  wcat = jnp.concatenate(
      [W_rel, W_root, W_gate, jnp.zeros((D, D - 3), jnp.float32)], axis=1)
  h, rrgt = _stage_b(aggp, cntb, xpad, W_l, b_l.reshape(1, D), W_r, wcat)

  r = rrgt[0]
  s2p = _stage_c(src2, dst2, r, zcnt)

  s2p3 = s2p.reshape(NC, NROW, 128)
  rho2d = rrgt[1].reshape(NROW, 128)
  g2d = rrgt[2].reshape(NROW, 128)
  brelb = jnp.broadcast_to(b_rel.reshape(1, 1), (1, 128))
  bgateb = jnp.broadcast_to(b_gate.reshape(1, 1), (1, 128))
  return _stage_d(s2p3, rho2d, g2d, h, brelb, bgateb)


# speedup vs baseline: 1.2584x; 1.1287x over previous
import functools

import jax
import jax.numpy as jnp
from jax import lax
from jax.experimental import pallas as pl
from jax.experimental.pallas import tpu as pltpu
from jax.experimental.pallas import tpu_sc as plsc

N = 10000
E = 320000
D = 128
NPAD = 10240
K = 8000
NC, NS = 2, 16
NW = NC * NS
CH = 128
NCHUNK = E // CH
ROWS_PT = NPAD // NS

_mesh = plsc.VectorSubcoreMesh(core_axis_name="c", subcore_axis_name="s")


EPAD = 2560 * CH
G = 4
CHK = G * CH
NCH3 = EPAD // CHK
NFIX = (EPAD // CH) // NW
EPT = NFIX * CH


def _agg_body(src_p, dst_p, x_hbm, zrow_hbm, zcnt_hbm, ones_hbm,
              agg_out, cnt_out, acc, cacc, sidx_f,
              rows0, rows1, d0, d1, d2, d3, ones_v,
              sg0, sg1, si0, si1, si2, si3, so0, so1, so2, so3):
  c = lax.axis_index("c")
  s = lax.axis_index("s")
  wid = s * NC + c
  row0 = s * ROWS_PT
  base = wid * EPT
  dbufs = (d0, d1, d2, d3)
  isems = (si0, si1, si2, si3)
  osems = (so0, so1, so2, so3)
  rbufs = (rows0, rows1)
  gsems = (sg0, sg1)
  pltpu.sync_copy(zrow_hbm, acc.at[pl.ds(row0, ROWS_PT)])
  pltpu.sync_copy(zcnt_hbm, cacc.at[pl.ds(row0, ROWS_PT)])
  pltpu.sync_copy(ones_hbm, ones_v)
  pltpu.sync_copy(src_p.at[pl.ds(base, EPT)], sidx_f)
  plsc.subcore_barrier()

  def gidx(i):
    return sidx_f.at[pl.ds(i * CH, CH)]

  def fire_gather(i, b4):
    pltpu.async_copy(x_hbm.at[gidx(i)], rbufs[b4], gsems[b4])

  def fire_didx(i, b4):
    pltpu.async_copy(dst_p.at[pl.ds(base + i * CH, CH)], dbufs[b4],
                     isems[b4])

  fire_didx(0, 0)
  fire_didx(1, 1)
  fire_gather(0, 0)
  fire_gather(1, 1)

  def group(j, carry):
    for b in range(4):
      i = 4 * j + b
      b2 = b % 2
      rbuf, dbuf = rbufs[b2], dbufs[b]
      pltpu.make_async_copy(dst_p.at[pl.ds(base, CH)], dbuf, isems[b]).wait()
      pltpu.make_async_copy(x_hbm.at[gidx(0)], rbuf, gsems[b2]).wait()
      pltpu.sync_copy(rbuf, acc.at[dbuf], add=True)
      pltpu.async_copy(ones_v, cacc.at[dbuf], osems[b], add=True)

      @pl.when(i + 2 < NFIX)
      def _():
        b4 = (b + 2) % 4

        @pl.when(i >= 2)
        def _():
          pltpu.make_async_copy(ones_v, cacc.at[d0], osems[b4]).wait()

        fire_didx(i + 2, b4)
        fire_gather(i + 2, b2)

    return carry

  lax.fori_loop(0, NFIX // 4, group, jnp.int32(0))
  for b4 in range(4):
    pltpu.make_async_copy(ones_v, cacc.at[d0], osems[b4]).wait()
  plsc.subcore_barrier()
  pltpu.sync_copy(acc.at[pl.ds(row0, ROWS_PT)],
                  agg_out.at[c, pl.ds(row0, ROWS_PT)])
  pltpu.sync_copy(cacc.at[pl.ds(row0, ROWS_PT)],
                  cnt_out.at[c, pl.ds(row0, ROWS_PT)])


_stage_a = functools.partial(
    pl.kernel,
    out_type=(jax.ShapeDtypeStruct((NC, NPAD, D), jnp.float32),
              jax.ShapeDtypeStruct((NC, NPAD), jnp.float32)),
    mesh=_mesh,
    scratch_types=[
        pltpu.VMEM_SHARED((NPAD, D), jnp.float32),
        pltpu.VMEM_SHARED((NPAD,), jnp.float32),
        pltpu.VMEM((EPT,), jnp.int32),
        pltpu.VMEM((CH, D), jnp.float32),
        pltpu.VMEM((CH, D), jnp.float32),
        pltpu.VMEM((CH,), jnp.int32),
        pltpu.VMEM((CH,), jnp.int32),
        pltpu.VMEM((CH,), jnp.int32),
        pltpu.VMEM((CH,), jnp.int32),
        pltpu.VMEM((CH,), jnp.float32),
    ] + [pltpu.SemaphoreType.DMA] * 10,
)(_agg_body)


NFC = NCH3 // NW


def _seg1_body(src2, dst2, r_hbm, zcnt_hbm,
               s2_out, sacc, r_local,
               vals0, vals1, s0, s1, d0, d1, d2, d3,
               sl0, sl1, si0, si1, si2, si3, sc0, sc1, sc2, sc3):
  c = lax.axis_index("c")
  s = lax.axis_index("s")
  wid = s * NC + c
  row0 = s * ROWS_PT
  base = wid * NFC
  sbufs = (s0, s1)
  dbufs = (d0, d1, d2, d3)
  vbufs = (vals0, vals1)
  lsems = (sl0, sl1)
  isems = (si0, si1, si2, si3)
  csems = (sc0, sc1, sc2, sc3)
  pltpu.sync_copy(zcnt_hbm, sacc.at[pl.ds(row0, ROWS_PT)])
  pltpu.sync_copy(r_hbm, r_local)

  def fire_sidx(i, b2):
    pltpu.async_copy(src2.at[base + i], sbufs[b2], lsems[b2])

  def fire_didx(i, b4):
    pltpu.async_copy(dst2.at[base + i], dbufs[b4], isems[b4])

  for b in range(2):
    fire_sidx(b, b)
    fire_didx(b, b)
  plsc.subcore_barrier()

  def fill(vbuf, sbuf):
    for k in range(CHK // 16):
      vs = sbuf[pl.ds(k * 16, 16)]
      vbuf[pl.ds(k * 16, 16)] = plsc.load_gather(r_local, [vs])

  def group(j, carry):
    for b in range(4):
      i = 4 * j + b
      b2 = b % 2
      vbuf, sbuf = vbufs[b2], sbufs[b2]
      pltpu.make_async_copy(src2.at[base], sbuf, lsems[b2]).wait()
      pltpu.make_async_copy(dst2.at[base], dbufs[b], isems[b]).wait()

      @pl.when(i >= 2)
      def _():
        pltpu.make_async_copy(vbuf, sacc.at[d0], csems[(b + 2) % 4]).wait()

      fill(vbuf, sbuf)

      @pl.when(i + 2 < NFC)
      def _():
        fire_sidx(i + 2, b2)
        fire_didx(i + 2, (b + 2) % 4)

      pltpu.async_copy(vbuf, sacc.at[dbufs[b]], csems[b], add=True)
    return carry

  lax.fori_loop(0, NFC // 4, group, jnp.int32(0))
  pltpu.make_async_copy(vals0, sacc.at[d0], csems[(NFC - 2) % 4]).wait()
  pltpu.make_async_copy(vals1, sacc.at[d0], csems[(NFC - 1) % 4]).wait()
  plsc.subcore_barrier()
  pltpu.sync_copy(sacc.at[pl.ds(row0, ROWS_PT)],
                  s2_out.at[c, pl.ds(row0, ROWS_PT)])


_stage_c = functools.partial(
    pl.kernel,
    out_type=jax.ShapeDtypeStruct((NC, NPAD), jnp.float32),
    mesh=_mesh,
    compiler_params=pltpu.CompilerParams(needs_layout_passes=False),
    scratch_types=[
        pltpu.VMEM_SHARED((NPAD,), jnp.float32),
        pltpu.VMEM((NPAD,), jnp.float32),
        pltpu.VMEM((CHK,), jnp.float32),
        pltpu.VMEM((CHK,), jnp.float32),
        pltpu.VMEM((CHK,), jnp.int32),
        pltpu.VMEM((CHK,), jnp.int32),
        pltpu.VMEM((CHK,), jnp.int32),
        pltpu.VMEM((CHK,), jnp.int32),
        pltpu.VMEM((CHK,), jnp.int32),
        pltpu.VMEM((CHK,), jnp.int32),
    ] + [pltpu.SemaphoreType.DMA] * 10,
)(_seg1_body)


BN = 1024


def _h_body(agg_ref, cntb_ref, x_ref, wl_ref, bl_ref, wr_ref, wcat_ref,
            h_ref, rrgt_ref):
  a = agg_ref[0] + agg_ref[1]
  mean = a / jnp.maximum(cntb_ref[...], 1.0)
  h = jnp.dot(mean, wl_ref[...], preferred_element_type=jnp.float32)
  h += jnp.dot(x_ref[...], wr_ref[...], preferred_element_type=jnp.float32)
  h = jnp.maximum(h + bl_ref[...], 0.0)
  h_ref[...] = h
  rrgt_ref[...] = lax.dot_general(
      wcat_ref[...], h, (((0,), (1,)), ((), ())),
      preferred_element_type=jnp.float32)


def _stage_b(aggp, cntb, xpad, wl, bl, wr, wcat):
  return pl.pallas_call(
      _h_body,
      grid=(NPAD // BN,),
      in_specs=[
          pl.BlockSpec((NC, BN, D), lambda i: (0, i, 0)),
          pl.BlockSpec((BN, D), lambda i: (i, 0)),
          pl.BlockSpec((BN, D), lambda i: (i, 0)),
          pl.BlockSpec((D, D), lambda i: (0, 0)),
          pl.BlockSpec((1, D), lambda i: (0, 0)),
          pl.BlockSpec((D, D), lambda i: (0, 0)),
          pl.BlockSpec((D, D), lambda i: (0, 0)),
      ],
      out_specs=[
          pl.BlockSpec((BN, D), lambda i: (i, 0)),
          pl.BlockSpec((D, BN), lambda i: (0, i)),
      ],
      out_shape=[
          jax.ShapeDtypeStruct((NPAD, D), jnp.float32),
          jax.ShapeDtypeStruct((D, NPAD), jnp.float32),
      ],
  )(aggp, cntb, xpad, wl, bl, wr, wcat)


NROW = NPAD // 128


def _readout_body(s2p_ref, rho_ref, g_ref, h_ref, brel_ref, bgate_ref,
                  out_ref, coef_ref):
  s2 = s2p_ref[0] + s2p_ref[1]
  score = jnp.tanh(s2 + rho_ref[...] + brel_ref[...])
  ub = lax.bitcast_convert_type(score, jnp.uint32)
  sgn = ub >> jnp.uint32(31)
  flip = jnp.where(sgn == jnp.uint32(1),
                   jnp.uint32(0xFFFFFFFF), jnp.uint32(0x80000000))
  key = ub ^ flip
  rows = lax.broadcasted_iota(jnp.int32, (NROW, 128), 0)
  cols = lax.broadcasted_iota(jnp.int32, (NROW, 128), 1)
  idx = rows * 128 + cols
  key = jnp.where(idx < N, key, jnp.uint32(0))

  def cnt_ge(m):
    return jnp.sum((key >= m).astype(jnp.int32))

  def bit_body(i, t):
    cand = t | (jnp.uint32(1) << (jnp.uint32(31) - i.astype(jnp.uint32)))
    return jnp.where(cnt_ge(cand) >= K, cand, t)

  t = lax.fori_loop(0, 32, bit_body, jnp.uint32(0))
  c_gt = cnt_ge(t + jnp.uint32(1))
  r_extra = K - c_gt
  ties = key == t

  def tie_cnt(j):
    return jnp.sum((ties & (idx <= j)).astype(jnp.int32))

  def bs_body(i, lohi):
    lo, hi = lohi
    mid = (lo + hi) // 2
    pred = tie_cnt(mid) >= r_extra
    return (jnp.where(pred, lo, mid + 1), jnp.where(pred, mid, hi))

  jstar, _ = lax.fori_loop(0, 14, bs_body,
                           (jnp.int32(0), jnp.int32(NPAD - 1)))
  sel = (key > t) | (ties & (idx <= jstar))

  gate = score * g_ref[...] + bgate_ref[...]
  gmax = jnp.max(jnp.where(sel, gate, -1e30))
  e = jnp.where(sel, jnp.exp(gate - gmax), 0.0)
  z = jnp.sum(e)
  coef_ref[...] = e * score / z

  def mv(rr, acc):
    crow = coef_ref[pl.ds(rr, 1), :]
    hblk = h_ref[pl.ds(rr * 128, 128), :]
    return acc + jnp.dot(crow, hblk, preferred_element_type=jnp.float32)

  out_ref[...] = lax.fori_loop(0, NROW, mv, jnp.zeros((1, D), jnp.float32))


def _stage_d(s2p3, rho2d, g2d, h, brelb, bgateb):
  return pl.pallas_call(
      _readout_body,
      out_shape=jax.ShapeDtypeStruct((1, D), jnp.float32),
      scratch_shapes=[pltpu.VMEM((NROW, 128), jnp.float32)],
  )(s2p3, rho2d, g2d, h, brelb, bgateb)


@jax.jit
def kernel(x, edge_index, W_l, b_l, W_r, W_rel, b_rel, W_root, W_gate,
           b_gate):
  src = edge_index[0]
  dst = edge_index[1]
  npad_e = EPAD - E
  fill = jnp.arange(npad_e, dtype=jnp.int32)
  src_p = jnp.concatenate([src, fill % N])
  dst_p = jnp.concatenate([dst, N + fill % (NPAD - N)])
  src2 = src_p.reshape(NCH3, CHK)
  dst2 = dst_p.reshape(NCH3, CHK)
  zrow = jnp.zeros((ROWS_PT, D), jnp.float32)
  zcnt = jnp.zeros((ROWS_PT,), jnp.float32)
  ones = jnp.ones((CH,), jnp.float32)

  aggp, cntp = _stage_a(src_p, dst_p, x, zrow, zcnt, ones)

  csum = cntp[0] + cntp[1]
  cntb = jnp.broadcast_to(csum[:, None], (NPAD, D))
  xpad = jnp.pad(x, ((0, NPAD - N), (0, 0)))
  wcat = jnp.concatenate(
      [W_rel, W_root, W_gate, jnp.zeros((D, D - 3), jnp.float32)], axis=1)
  h, rrgt = _stage_b(aggp, cntb, xpad, W_l, b_l.reshape(1, D), W_r, wcat)

  r = rrgt[0]
  s2p = _stage_c(src2, dst2, r, zcnt)

  s2p3 = s2p.reshape(NC, NROW, 128)
  rho2d = rrgt[1].reshape(NROW, 128)
  g2d = rrgt[2].reshape(NROW, 128)
  brelb = jnp.broadcast_to(b_rel.reshape(1, 1), (1, 128))
  bgateb = jnp.broadcast_to(b_gate.reshape(1, 1), (1, 128))
  return _stage_d(s2p3, rho2d, g2d, h, brelb, bgateb)


# speedup vs baseline: 1.3338x; 1.0599x over previous
import functools

import jax
import jax.numpy as jnp
from jax import lax
from jax.experimental import pallas as pl
from jax.experimental.pallas import tpu as pltpu
from jax.experimental.pallas import tpu_sc as plsc

N = 10000
E = 320000
D = 128
NPAD = 10240
K = 8000
NC, NS = 2, 16
NW = NC * NS
CH = 128
NCHUNK = E // CH
ROWS_PT = NPAD // NS

_mesh = plsc.VectorSubcoreMesh(core_axis_name="c", subcore_axis_name="s")


EPAD = 2560 * CH
G = 4
CHK = G * CH
NCH3 = EPAD // CHK
NFIX = (EPAD // CH) // NW
EPT = NFIX * CH


def _agg_body(src_p, dst_p, x_hbm, zrow_hbm, zcnt_hbm, ones_hbm,
              agg_out, cnt_out, acc, cacc, sidx_f,
              rows0, rows1, d0, d1, d2, d3, ones_v,
              sg0, sg1, si0, si1, si2, si3, so0, so1, so2, so3):
  c = lax.axis_index("c")
  s = lax.axis_index("s")
  wid = s * NC + c
  row0 = s * ROWS_PT
  base = wid * EPT
  dbufs = (d0, d1, d2, d3)
  isems = (si0, si1, si2, si3)
  osems = (so0, so1, so2, so3)
  rbufs = (rows0, rows1)
  gsems = (sg0, sg1)
  pltpu.sync_copy(zrow_hbm, acc.at[pl.ds(row0, ROWS_PT)])
  pltpu.sync_copy(zcnt_hbm, cacc.at[pl.ds(row0, ROWS_PT)])
  pltpu.sync_copy(ones_hbm, ones_v)
  pltpu.sync_copy(src_p.at[pl.ds(base, EPT)], sidx_f)
  plsc.subcore_barrier()

  def gidx(i):
    return sidx_f.at[pl.ds(i * CH, CH)]

  def fire_gather(i, b4):
    pltpu.async_copy(x_hbm.at[gidx(i)], rbufs[b4], gsems[b4])

  def fire_didx(i, b4):
    pltpu.async_copy(dst_p.at[pl.ds(base + i * CH, CH)], dbufs[b4],
                     isems[b4])

  fire_didx(0, 0)
  fire_didx(1, 1)
  fire_gather(0, 0)
  fire_gather(1, 1)

  def group(j, carry):
    for b in range(4):
      i = 4 * j + b
      b2 = b % 2
      rbuf, dbuf = rbufs[b2], dbufs[b]
      pltpu.make_async_copy(dst_p.at[pl.ds(base, CH)], dbuf, isems[b]).wait()
      pltpu.make_async_copy(x_hbm.at[gidx(0)], rbuf, gsems[b2]).wait()
      pltpu.sync_copy(rbuf, acc.at[dbuf], add=True)
      pltpu.async_copy(ones_v, cacc.at[dbuf], osems[b], add=True)

      @pl.when(i + 2 < NFIX)
      def _():
        b4 = (b + 2) % 4

        @pl.when(i >= 2)
        def _():
          pltpu.make_async_copy(ones_v, cacc.at[d0], osems[b4]).wait()

        fire_didx(i + 2, b4)
        fire_gather(i + 2, b2)

    return carry

  lax.fori_loop(0, NFIX // 4, group, jnp.int32(0))
  for b4 in range(4):
    pltpu.make_async_copy(ones_v, cacc.at[d0], osems[b4]).wait()
  plsc.subcore_barrier()
  pltpu.sync_copy(acc.at[pl.ds(row0, ROWS_PT)],
                  agg_out.at[c, pl.ds(row0, ROWS_PT)])
  pltpu.sync_copy(cacc.at[pl.ds(row0, ROWS_PT)],
                  cnt_out.at[c, pl.ds(row0, ROWS_PT)])


_stage_a = functools.partial(
    pl.kernel,
    out_type=(jax.ShapeDtypeStruct((NC, NPAD, D), jnp.float32),
              jax.ShapeDtypeStruct((NC, NPAD), jnp.float32)),
    mesh=_mesh,
    scratch_types=[
        pltpu.VMEM_SHARED((NPAD, D), jnp.float32),
        pltpu.VMEM_SHARED((NPAD,), jnp.float32),
        pltpu.VMEM((EPT,), jnp.int32),
        pltpu.VMEM((CH, D), jnp.float32),
        pltpu.VMEM((CH, D), jnp.float32),
        pltpu.VMEM((CH,), jnp.int32),
        pltpu.VMEM((CH,), jnp.int32),
        pltpu.VMEM((CH,), jnp.int32),
        pltpu.VMEM((CH,), jnp.int32),
        pltpu.VMEM((CH,), jnp.float32),
    ] + [pltpu.SemaphoreType.DMA] * 10,
)(_agg_body)


NFC = NCH3 // NW


def _seg1_body(src2, dst2, r_hbm, zcnt_hbm,
               s2_out, sacc, r_local,
               vals0, vals1, s0, s1, d0, d1, d2, d3,
               sl0, sl1, si0, si1, si2, si3, sc0, sc1, sc2, sc3):
  c = lax.axis_index("c")
  s = lax.axis_index("s")
  wid = s * NC + c
  row0 = s * ROWS_PT
  base = wid * NFC
  sbufs = (s0, s1)
  dbufs = (d0, d1, d2, d3)
  vbufs = (vals0, vals1)
  lsems = (sl0, sl1)
  isems = (si0, si1, si2, si3)
  csems = (sc0, sc1, sc2, sc3)
  pltpu.sync_copy(zcnt_hbm, sacc.at[pl.ds(row0, ROWS_PT)])
  pltpu.sync_copy(r_hbm, r_local)

  def fire_sidx(i, b2):
    pltpu.async_copy(src2.at[base + i], sbufs[b2], lsems[b2])

  def fire_didx(i, b4):
    pltpu.async_copy(dst2.at[base + i], dbufs[b4], isems[b4])

  for b in range(2):
    fire_sidx(b, b)
    fire_didx(b, b)
  plsc.subcore_barrier()

  def fill(vbuf, sbuf):
    for k in range(CHK // 16):
      vs = sbuf[pl.ds(k * 16, 16)]
      vbuf[pl.ds(k * 16, 16)] = plsc.load_gather(r_local, [vs])

  def group(j, carry):
    for b in range(4):
      i = 4 * j + b
      b2 = b % 2
      vbuf, sbuf = vbufs[b2], sbufs[b2]
      pltpu.make_async_copy(src2.at[base], sbuf, lsems[b2]).wait()
      pltpu.make_async_copy(dst2.at[base], dbufs[b], isems[b]).wait()

      @pl.when(i >= 2)
      def _():
        pltpu.make_async_copy(vbuf, sacc.at[d0], csems[(b + 2) % 4]).wait()

      fill(vbuf, sbuf)

      @pl.when(i + 2 < NFC)
      def _():
        fire_sidx(i + 2, b2)
        fire_didx(i + 2, (b + 2) % 4)

      pltpu.async_copy(vbuf, sacc.at[dbufs[b]], csems[b], add=True)
    return carry

  lax.fori_loop(0, NFC // 4, group, jnp.int32(0))
  pltpu.make_async_copy(vals0, sacc.at[d0], csems[(NFC - 2) % 4]).wait()
  pltpu.make_async_copy(vals1, sacc.at[d0], csems[(NFC - 1) % 4]).wait()
  plsc.subcore_barrier()
  pltpu.sync_copy(sacc.at[pl.ds(row0, ROWS_PT)],
                  s2_out.at[c, pl.ds(row0, ROWS_PT)])


_stage_c = functools.partial(
    pl.kernel,
    out_type=jax.ShapeDtypeStruct((NC, NPAD), jnp.float32),
    mesh=_mesh,
    compiler_params=pltpu.CompilerParams(needs_layout_passes=False),
    scratch_types=[
        pltpu.VMEM_SHARED((NPAD,), jnp.float32),
        pltpu.VMEM((NPAD,), jnp.float32),
        pltpu.VMEM((CHK,), jnp.float32),
        pltpu.VMEM((CHK,), jnp.float32),
        pltpu.VMEM((CHK,), jnp.int32),
        pltpu.VMEM((CHK,), jnp.int32),
        pltpu.VMEM((CHK,), jnp.int32),
        pltpu.VMEM((CHK,), jnp.int32),
        pltpu.VMEM((CHK,), jnp.int32),
        pltpu.VMEM((CHK,), jnp.int32),
    ] + [pltpu.SemaphoreType.DMA] * 10,
)(_seg1_body)


BN = 1024


def _h_body(agg_ref, cntb_ref, x_ref, wl_ref, bl_ref, wr_ref, wcat_ref,
            h_ref, rrgt_ref):
  a = agg_ref[0] + agg_ref[1]
  mean = a / jnp.maximum(cntb_ref[...], 1.0)
  h = jnp.dot(mean, wl_ref[...], preferred_element_type=jnp.float32)
  h += jnp.dot(x_ref[...], wr_ref[...], preferred_element_type=jnp.float32)
  h = jnp.maximum(h + bl_ref[...], 0.0)
  h_ref[...] = h
  rrgt_ref[...] = lax.dot_general(
      wcat_ref[...], h, (((0,), (1,)), ((), ())),
      preferred_element_type=jnp.float32)


def _stage_b(aggp, cntb, xpad, wl, bl, wr, wcat):
  return pl.pallas_call(
      _h_body,
      grid=(NPAD // BN,),
      in_specs=[
          pl.BlockSpec((NC, BN, D), lambda i: (0, i, 0)),
          pl.BlockSpec((BN, 1), lambda i: (i, 0)),
          pl.BlockSpec((BN, D), lambda i: (i, 0)),
          pl.BlockSpec((D, D), lambda i: (0, 0)),
          pl.BlockSpec((1, D), lambda i: (0, 0)),
          pl.BlockSpec((D, D), lambda i: (0, 0)),
          pl.BlockSpec((D, D), lambda i: (0, 0)),
      ],
      out_specs=[
          pl.BlockSpec((BN, D), lambda i: (i, 0)),
          pl.BlockSpec((D, BN), lambda i: (0, i)),
      ],
      out_shape=[
          jax.ShapeDtypeStruct((NPAD, D), jnp.float32),
          jax.ShapeDtypeStruct((D, NPAD), jnp.float32),
      ],
  )(aggp, cntb, xpad, wl, bl, wr, wcat)


NROW = NPAD // 128


def _readout_body(s2p_ref, rho_ref, g_ref, h_ref, brel_ref, bgate_ref,
                  out_ref, coef_ref):
  s2 = s2p_ref[0] + s2p_ref[1]
  score = jnp.tanh(s2 + rho_ref[...] + brel_ref[...])
  ub = lax.bitcast_convert_type(score, jnp.uint32)
  sgn = ub >> jnp.uint32(31)
  flip = jnp.where(sgn == jnp.uint32(1),
                   jnp.uint32(0xFFFFFFFF), jnp.uint32(0x80000000))
  key = ub ^ flip
  rows = lax.broadcasted_iota(jnp.int32, (NROW, 128), 0)
  cols = lax.broadcasted_iota(jnp.int32, (NROW, 128), 1)
  idx = rows * 128 + cols
  key = jnp.where(idx < N, key, jnp.uint32(0))

  def cnt_ge(m):
    return jnp.sum((key >= m).astype(jnp.int32))

  def bit_body(i, t):
    cand = t | (jnp.uint32(1) << (jnp.uint32(31) - i.astype(jnp.uint32)))
    return jnp.where(cnt_ge(cand) >= K, cand, t)

  t = lax.fori_loop(0, 32, bit_body, jnp.uint32(0))
  c_gt = cnt_ge(t + jnp.uint32(1))
  r_extra = K - c_gt
  ties = key == t

  def tie_cnt(j):
    return jnp.sum((ties & (idx <= j)).astype(jnp.int32))

  def bs_body(i, lohi):
    lo, hi = lohi
    mid = (lo + hi) // 2
    pred = tie_cnt(mid) >= r_extra
    return (jnp.where(pred, lo, mid + 1), jnp.where(pred, mid, hi))

  jstar, _ = lax.fori_loop(0, 14, bs_body,
                           (jnp.int32(0), jnp.int32(NPAD - 1)))
  sel = (key > t) | (ties & (idx <= jstar))

  gate = score * g_ref[...] + bgate_ref[...]
  gmax = jnp.max(jnp.where(sel, gate, -1e30))
  e = jnp.where(sel, jnp.exp(gate - gmax), 0.0)
  z = jnp.sum(e)
  coef_ref[...] = e * score / z

  h3 = h_ref[...].reshape(NROW, 128, D)
  coef3 = coef_ref[...].reshape(NROW, 1, 128)
  prods = lax.dot_general(coef3, h3, (((2,), (1,)), ((0,), (0,))),
                          preferred_element_type=jnp.float32)
  out_ref[...] = jnp.sum(prods, axis=0)


def _stage_d(s2p3, rho2d, g2d, h, brelb, bgateb):
  return pl.pallas_call(
      _readout_body,
      out_shape=jax.ShapeDtypeStruct((1, D), jnp.float32),
      scratch_shapes=[pltpu.VMEM((NROW, 128), jnp.float32)],
  )(s2p3, rho2d, g2d, h, brelb, bgateb)


@jax.jit
def kernel(x, edge_index, W_l, b_l, W_r, W_rel, b_rel, W_root, W_gate,
           b_gate):
  src = edge_index[0]
  dst = edge_index[1]
  npad_e = EPAD - E
  fill = jnp.arange(npad_e, dtype=jnp.int32)
  src_p = jnp.concatenate([src, fill % N])
  dst_p = jnp.concatenate([dst, N + fill % (NPAD - N)])
  src2 = src_p.reshape(NCH3, CHK)
  dst2 = dst_p.reshape(NCH3, CHK)
  zrow = jnp.zeros((ROWS_PT, D), jnp.float32)
  zcnt = jnp.zeros((ROWS_PT,), jnp.float32)
  ones = jnp.ones((CH,), jnp.float32)

  aggp, cntp = _stage_a(src_p, dst_p, x, zrow, zcnt, ones)

  csum = cntp[0] + cntp[1]
  cntb = csum[:, None]
  xpad = jnp.pad(x, ((0, NPAD - N), (0, 0)))
  wcat = jnp.concatenate(
      [W_rel, W_root, W_gate, jnp.zeros((D, D - 3), jnp.float32)], axis=1)
  h, rrgt = _stage_b(aggp, cntb, xpad, W_l, b_l.reshape(1, D), W_r, wcat)

  r = rrgt[0]
  s2p = _stage_c(src2, dst2, r, zcnt)

  s2p3 = s2p.reshape(NC, NROW, 128)
  rho2d = rrgt[1].reshape(NROW, 128)
  g2d = rrgt[2].reshape(NROW, 128)
  brelb = jnp.broadcast_to(b_rel.reshape(1, 1), (1, 128))
  bgateb = jnp.broadcast_to(b_gate.reshape(1, 1), (1, 128))
  return _stage_d(s2p3, rho2d, g2d, h, brelb, bgateb)
